# baseline (speedup 1.0000x reference)
import functools

import numpy as np
import jax
import jax.numpy as jnp
from jax import lax
from jax.experimental import pallas as pl
from jax.experimental.pallas import tpu as pltpu
from jax.experimental.pallas import tpu_sc as plsc

_N = 262144
_NUM_LEVELS = 16
_LOG2_HASH = 21
_TBL = 1 << _LOG2_HASH
_MASK = _TBL - 1
_P0, _P1, _P2 = 73856093, 19349663, 83492791
_GROWTH = np.exp((np.log(2048.0) - np.log(16.0)) / (_NUM_LEVELS - 1))
_RES = np.floor(16.0 * _GROWTH ** np.arange(_NUM_LEVELS)).astype(np.float32)
_OFFS = [(0, 0, 0), (0, 0, 1), (0, 1, 0), (0, 1, 1),
         (1, 0, 0), (1, 0, 1), (1, 1, 0), (1, 1, 1)]

_NW = 32
_PPW = _N // _NW
_C = 1024
_NCHUNK = _PPW // _C
_G = _C // 16


@functools.cache
def _build():
    mesh = plsc.VectorSubcoreMesh(core_axis_name="c", subcore_axis_name="s")

    @functools.partial(
        pl.kernel,
        out_type=jax.ShapeDtypeStruct((_N, 2 * _NUM_LEVELS), jnp.float32),
        mesh=mesh,
        compiler_params=pltpu.CompilerParams(
            needs_layout_passes=False, use_tc_tiling_on_sc=False),
        scratch_types=[
            pltpu.VMEM((3, _C), jnp.float32),
            pltpu.VMEM((3, _C), jnp.float32),
            pltpu.VMEM((3, _C), jnp.float32),
            pltpu.VMEM((8 * _C,), jnp.int32),
            pltpu.VMEM((8 * _C,), jnp.int32),
            pltpu.VMEM((8 * _C,), jnp.int32),
            pltpu.VMEM((8 * _C,), jnp.int32),
            pltpu.VMEM((8 * _C,), jnp.float32),
            pltpu.VMEM((8 * _C,), jnp.float32),
            pltpu.VMEM((8 * _C,), jnp.float32),
            pltpu.VMEM((8 * _C,), jnp.float32),
            pltpu.VMEM((_C, 2 * _NUM_LEVELS), jnp.float32),
            pltpu.SemaphoreType.DMA,
            pltpu.SemaphoreType.DMA,
            pltpu.SemaphoreType.DMA,
            pltpu.SemaphoreType.DMA,
        ],
    )
    def sc_embed(xt_hbm, tab_hbm, out_hbm, xn_v, w_a, w_b,
                 i0_a, i1_a, i0_b, i1_b, r0_a, r1_a, r0_b, r1_b,
                 out_v, sem0_a, sem1_a, sem0_b, sem1_b):
        wid = lax.axis_index("s") * 2 + lax.axis_index("c")
        ii = lax.broadcasted_iota(jnp.int32, (16,), 0)
        bufs = [(i0_a, i1_a, r0_a, r1_a, w_a, sem0_a, sem1_a),
                (i0_b, i1_b, r0_b, r1_b, w_b, sem0_b, sem1_b)]

        def run_chunk(c, carry):
            base = wid * np.int32(_PPW) + c * np.int32(_C)
            pltpu.sync_copy(xt_hbm.at[:, pl.ds(base, _C)], xn_v)

            def norm_g(g, carry):
                s = pl.ds(g * np.int32(16), 16)
                for d in range(3):
                    v = (xn_v[d, s] + 2.0) / 4.0
                    xn_v[d, s] = jnp.minimum(jnp.maximum(v, 0.0), 1.0)
                return carry
            lax.fori_loop(jnp.int32(0), jnp.int32(_G), norm_g, 0)

            def pass1(l):
                i0_v, i1_v, _r0, _r1, w_v, _s0, _s1 = bufs[l % 2]
                res = float(_RES[l])
                off2 = l * 2 * _TBL

                def g_body(g, carry):
                    s = pl.ds(g * np.int32(16), 16)
                    sx = xn_v[0, s] * res
                    sy = xn_v[1, s] * res
                    sz = xn_v[2, s] * res
                    ix = sx.astype(jnp.int32)
                    iy = sy.astype(jnp.int32)
                    iz = sz.astype(jnp.int32)
                    w_v[0, s] = sx - ix.astype(jnp.float32)
                    w_v[1, s] = sy - iy.astype(jnp.float32)
                    w_v[2, s] = sz - iz.astype(jnp.float32)
                    ax = (ix * _P0, (ix + 1) * _P0)
                    ay = (iy * _P1, (iy + 1) * _P1)
                    az = (iz * _P2, (iz + 1) * _P2)
                    gbase = g * np.int32(16)
                    for j, (ox, oy, oz) in enumerate(_OFFS):
                        h2 = (((ax[ox] ^ ay[oy] ^ az[oz]) & _MASK) * 2) + off2
                        sj = pl.ds(gbase + np.int32(j * _C), 16)
                        i0_v[sj] = h2
                        i1_v[sj] = h2 + 1
                    return carry
                lax.fori_loop(jnp.int32(0), jnp.int32(_G), g_body, 0)

            def start_gather(l):
                i0_v, i1_v, r0_v, r1_v, _w, s0, s1 = bufs[l % 2]
                h0 = pltpu.async_copy(tab_hbm.at[i0_v], r0_v, s0)
                h1 = pltpu.async_copy(tab_hbm.at[i1_v], r1_v, s1)
                return h0, h1

            def pass2(l):
                _i0, _i1, r0_v, r1_v, w_v, _s0, _s1 = bufs[l % 2]
                col0 = jnp.full((16,), 2 * l, jnp.int32)
                col1 = jnp.full((16,), 2 * l + 1, jnp.int32)

                def g_body(g, carry):
                    s = pl.ds(g * np.int32(16), 16)
                    wx = w_v[0, s]
                    wy = w_v[1, s]
                    wz = w_v[2, s]
                    ux = 1.0 - wx
                    uy = 1.0 - wy
                    uz = 1.0 - wz
                    wyz = ((uy * uz, uy * wz), (wy * uz, wy * wz))
                    gbase = g * np.int32(16)
                    acc0 = None
                    acc1 = None
                    for j, (ox, oy, oz) in enumerate(_OFFS):
                        wj = (ux if ox == 0 else wx) * wyz[oy][oz]
                        sj = pl.ds(gbase + np.int32(j * _C), 16)
                        f0 = r0_v[sj]
                        f1 = r1_v[sj]
                        acc0 = wj * f0 if acc0 is None else acc0 + wj * f0
                        acc1 = wj * f1 if acc1 is None else acc1 + wj * f1
                    rb = ii + gbase
                    plsc.store_scatter(out_v, [rb, col0], acc0)
                    plsc.store_scatter(out_v, [rb, col1], acc1)
                    return carry
                lax.fori_loop(jnp.int32(0), jnp.int32(_G), g_body, 0)

            pass1(0)
            h = start_gather(0)
            for l in range(_NUM_LEVELS):
                if l + 1 < _NUM_LEVELS:
                    pass1(l + 1)
                    h_next = start_gather(l + 1)
                else:
                    h_next = None
                h[0].wait()
                h[1].wait()
                pass2(l)
                h = h_next
            pltpu.sync_copy(out_v, out_hbm.at[pl.ds(base, _C), :])
            return carry
        lax.fori_loop(jnp.int32(0), jnp.int32(_NCHUNK), run_chunk, 0)

    return sc_embed


def kernel(x, tables):
    x_t = x.astype(jnp.float32).T
    tab = tables.astype(jnp.float32).reshape(-1)
    return _build()(x_t, tab)
